# baseline (speedup 1.0000x reference)
import functools

import jax
import jax.numpy as jnp
from jax import lax
from jax.experimental import pallas as pl
from jax.experimental.pallas import tpu as pltpu
from jax.experimental.pallas import tpu_sc as plsc

N = 10000
NPAD = 10240
E = 320000
F = 128
NC = 2
NS = 16
NW = NC * NS
EPW = E // NW
CH = 128
NCHTOT = E // CH
MAXCH = 84
NB = 2
RPS = NPAD // NS

_SC_MESH = plsc.VectorSubcoreMesh(core_axis_name="c", subcore_axis_name="s")


def _degrees_body(src_hbm, dst_hbm, out_hbm, idx_v, cnt_v):
    c = lax.axis_index("c")
    s = lax.axis_index("s")
    wid = c * NS + s
    ones = jnp.ones((16,), jnp.float32)
    zeros = jnp.zeros((16,), jnp.float32)
    for kind, edge_hbm in ((0, src_hbm), (1, dst_hbm)):
        def zero_step(i, _):
            cnt_v[pl.ds(i * 16, 16)] = zeros
            return 0
        lax.fori_loop(0, NPAD // 16, zero_step, 0)
        pltpu.sync_copy(edge_hbm.at[pl.ds(wid * EPW, EPW)], idx_v)

        def count_step(j, _):
            idx = idx_v[pl.ds(j * 16, 16)]
            plsc.addupdate_scatter(cnt_v, [idx], ones)
            return 0
        lax.fori_loop(0, EPW // 16, count_step, 0)
        pltpu.sync_copy(cnt_v, out_hbm.at[kind, wid])


_sc_degrees = pl.kernel(
    _degrees_body,
    out_type=jax.ShapeDtypeStruct((2, NW, NPAD), jnp.float32),
    mesh=_SC_MESH,
    compiler_params=pltpu.CompilerParams(needs_layout_passes=False),
    scratch_types=[
        pltpu.VMEM((EPW,), jnp.int32),
        pltpu.VMEM((NPAD,), jnp.float32),
    ],
)


def _agg_body(g_hbm, src_hbm, dst_hbm, zeros_hbm, out_hbm,
              s0, s1, d0, d1, d2, rows, acc_sh, gsems, isems, dsems, ssems):
    c = lax.axis_index("c")
    s = lax.axis_index("s")
    wid = c * NS + s
    lo = ((wid * NCHTOT) // NW) // 8 * 8
    hi = (((wid + 1) * NCHTOT) // NW) // 8 * 8
    hi = jnp.where(wid == NW - 1, NCHTOT, hi)
    n = hi - lo

    scur = (s0, s1)
    dcur = (d0, d1, d2)

    def ipref(cc, sb, db):
        pltpu.async_copy(src_hbm.at[pl.ds((lo + cc) * CH, CH)],
                         scur[sb], isems[sb])
        pltpu.async_copy(dst_hbm.at[pl.ds((lo + cc) * CH, CH)],
                         dcur[db], dsems[db])

    def iwait(cc, sb):
        pltpu.make_async_copy(src_hbm.at[pl.ds((lo + cc) * CH, CH)],
                              scur[sb], isems[sb]).wait()

    def dwait(cc, db):
        pltpu.make_async_copy(dst_hbm.at[pl.ds((lo + cc) * CH, CH)],
                              dcur[db], dsems[db]).wait()

    def gstart(sb, rb):
        pltpu.async_copy(g_hbm.at[scur[sb]], rows.at[rb], gsems[rb])

    def gwait(sb, rb):
        pltpu.make_async_copy(g_hbm.at[scur[sb]], rows.at[rb],
                              gsems[rb]).wait()

    def sstart(rb, db):
        pltpu.async_copy(rows.at[rb], acc_sh.at[dcur[db]], ssems[db % 2],
                         add=True)

    def swait(rb, db):
        pltpu.make_async_copy(rows.at[rb], acc_sh.at[dcur[db]],
                              ssems[db % 2]).wait()

    pltpu.sync_copy(zeros_hbm, acc_sh.at[pl.ds(s * RPS, RPS)])
    plsc.subcore_barrier()

    ipref(0, 0, 0)
    ipref(1, 1, 1)
    iwait(0, 0)
    gstart(0, 0)

    UNROLL = 6

    def outer(i, _):
        for k in range(UNROLL):
            cc = i * UNROLL + k
            rb = k % NB
            sb = k % 2
            db = k % 3

            @pl.when((cc >= 1) & (cc <= n))
            def _():
                swait((k + 1) % NB, (k + 2) % 3)

            @pl.when(cc + 1 < n)
            def _():
                iwait(cc + 1, (k + 1) % 2)
                gstart((k + 1) % 2, (k + 1) % NB)

            @pl.when(cc < n)
            def _():
                gwait(sb, rb)

            @pl.when(cc + 2 < n)
            def _():
                ipref(cc + 2, sb, (k + 2) % 3)

            @pl.when(cc < n)
            def _():
                dwait(cc, db)
                sstart(rb, db)
        return 0
    lax.fori_loop(0, MAXCH // UNROLL + 1, outer, 0)

    plsc.subcore_barrier()
    pltpu.sync_copy(acc_sh.at[pl.ds(s * RPS, RPS)],
                    out_hbm.at[c, pl.ds(s * RPS, RPS)])


_sc_aggregate = pl.kernel(
    _agg_body,
    out_type=jax.ShapeDtypeStruct((NC, NPAD, F), jnp.float32),
    mesh=_SC_MESH,
    scratch_types=[
        pltpu.VMEM((CH,), jnp.int32),
        pltpu.VMEM((CH,), jnp.int32),
        pltpu.VMEM((CH,), jnp.int32),
        pltpu.VMEM((CH,), jnp.int32),
        pltpu.VMEM((CH,), jnp.int32),
        pltpu.VMEM((NB, CH, F), jnp.float32),
        pltpu.VMEM_SHARED((NPAD, F), jnp.float32),
        [pltpu.SemaphoreType.DMA] * NB,
        [pltpu.SemaphoreType.DMA] * 2,
        [pltpu.SemaphoreType.DMA] * 3,
        [pltpu.SemaphoreType.DMA] * 2,
    ],
)


def _norms_body(cnt_ref, out_ref):
    deg = jnp.sum(cnt_ref[...], axis=1)
    out_ref[...] = jnp.where(deg > 0, lax.rsqrt(deg), 0.0)


def _tc_norms(counts4):
    return pl.pallas_call(
        _norms_body,
        out_shape=jax.ShapeDtypeStruct((2, NPAD // F, F), jnp.float32),
    )(counts4)


BM = 1024
GRID_M = NPAD // BM


def _mm1_body(x_ref, w_ref, no_ref, out_ref):
    out_ref[...] = jnp.dot(
        x_ref[...], w_ref[...], preferred_element_type=jnp.float32
    ) * no_ref[...]


def _tc_mm1(x, W1, norm_out):
    return pl.pallas_call(
        _mm1_body,
        grid=(GRID_M,),
        in_specs=[
            pl.BlockSpec((BM, F), lambda m: (m, 0)),
            pl.BlockSpec((F, F), lambda m: (0, 0)),
            pl.BlockSpec((BM, 1), lambda m: (m, 0)),
        ],
        out_specs=pl.BlockSpec((BM, F), lambda m: (m, 0)),
        out_shape=jax.ShapeDtypeStruct((NPAD, F), jnp.float32),
    )(x, W1, norm_out)


def _mm2_body(p_ref, ni_ref, b_ref, w_ref, no_ref, out_ref):
    agg = p_ref[0].astype(jnp.float32) + p_ref[1].astype(jnp.float32)
    h = jnp.maximum(agg * ni_ref[...] + b_ref[...], 0.0)
    out_ref[...] = jnp.dot(
        h, w_ref[...], preferred_element_type=jnp.float32
    ) * no_ref[...]


def _tc_mm2(part, norm_in, b1, W2, norm_out):
    return pl.pallas_call(
        _mm2_body,
        grid=(GRID_M,),
        in_specs=[
            pl.BlockSpec((NC, BM, F), lambda m: (0, m, 0)),
            pl.BlockSpec((BM, 1), lambda m: (m, 0)),
            pl.BlockSpec((1, F), lambda m: (0, 0)),
            pl.BlockSpec((F, F), lambda m: (0, 0)),
            pl.BlockSpec((BM, 1), lambda m: (m, 0)),
        ],
        out_specs=pl.BlockSpec((BM, F), lambda m: (m, 0)),
        out_shape=jax.ShapeDtypeStruct((NPAD, F), jnp.float32),
    )(part, norm_in, b1.reshape(1, F), W2, norm_out)


def _final_body(p_ref, ni_ref, b_ref, out_ref):
    m = pl.program_id(0)
    agg = p_ref[0].astype(jnp.float32) + p_ref[1].astype(jnp.float32)
    h = jnp.maximum(agg * ni_ref[...] + b_ref[...], 0.0)
    rows = lax.broadcasted_iota(jnp.int32, (BM, 1), 0) + m * BM
    h = jnp.where(rows < N, h, 0.0)
    part = jnp.sum(h, axis=0, keepdims=True) * (1.0 / N)

    @pl.when(m == 0)
    def _():
        out_ref[...] = part

    @pl.when(m > 0)
    def _():
        out_ref[...] += part


def _tc_final(part, norm_in, b2):
    return pl.pallas_call(
        _final_body,
        grid=(GRID_M,),
        in_specs=[
            pl.BlockSpec((NC, BM, F), lambda m: (0, m, 0)),
            pl.BlockSpec((BM, 1), lambda m: (m, 0)),
            pl.BlockSpec((1, F), lambda m: (0, 0)),
        ],
        out_specs=pl.BlockSpec((1, F), lambda m: (0, 0)),
        out_shape=jax.ShapeDtypeStruct((1, F), jnp.float32),
    )(part, norm_in, b2.reshape(1, F))


@jax.jit
def kernel(node_feats, edge_index, W1, b1, W2, b2):
    src = edge_index[0].astype(jnp.int32)
    dst = edge_index[1].astype(jnp.int32)

    counts = _sc_degrees(src, dst)
    norms = _tc_norms(counts.reshape(2, NW, NPAD // F, F))
    norm_out = norms[0].reshape(NPAD, 1)
    norm_in = norms[1].reshape(NPAD, 1)

    xp = jnp.zeros((NPAD, F), jnp.float32).at[:N].set(node_feats)
    zrows = jnp.zeros((RPS, F), jnp.float32)

    g1 = _tc_mm1(xp, W1, norm_out)
    p1 = _sc_aggregate(g1, src, dst, zrows)
    g2 = _tc_mm2(p1, norm_in, b1, W2, norm_out)
    p2 = _sc_aggregate(g2, src, dst, zrows)
    return _tc_final(p2, norm_in, b2)


# speedup vs baseline: 1.1000x; 1.1000x over previous
import functools

import jax
import jax.numpy as jnp
from jax import lax
from jax.experimental import pallas as pl
from jax.experimental.pallas import tpu as pltpu
from jax.experimental.pallas import tpu_sc as plsc

N = 10000
NPAD = 10240
E = 320000
F = 128
NC = 2
NS = 16
NW = NC * NS
EPW = E // NW
CH = 64
NCHTOT = E // CH
MAXCH = 160
NB = 4
RPS = NPAD // NS

_SC_MESH = plsc.VectorSubcoreMesh(core_axis_name="c", subcore_axis_name="s")


def _degrees_body(src_hbm, dst_hbm, zflat_hbm, out_hbm, idx_v, cnt_v):
    c = lax.axis_index("c")
    s = lax.axis_index("s")
    wid = c * NS + s
    ones = jnp.ones((16,), jnp.float32)
    for kind, edge_hbm in ((0, src_hbm), (1, dst_hbm)):
        pltpu.sync_copy(zflat_hbm, cnt_v)
        pltpu.sync_copy(edge_hbm.at[pl.ds(wid * EPW, EPW)], idx_v)

        def count_step(j, _):
            for q in range(4):
                idx = idx_v[pl.ds(j * 64 + q * 16, 16)]
                plsc.addupdate_scatter(cnt_v, [idx], ones)
            return 0
        lax.fori_loop(0, EPW // 64, count_step, 0)
        for r in range(EPW // 64 * 64, EPW, 16):
            plsc.addupdate_scatter(cnt_v, [idx_v[pl.ds(r, 16)]], ones)
        pltpu.sync_copy(cnt_v, out_hbm.at[kind, wid])


_sc_degrees = pl.kernel(
    _degrees_body,
    out_type=jax.ShapeDtypeStruct((2, NW, NPAD), jnp.float32),
    mesh=_SC_MESH,
    compiler_params=pltpu.CompilerParams(needs_layout_passes=False),
    scratch_types=[
        pltpu.VMEM((EPW,), jnp.int32),
        pltpu.VMEM((NPAD,), jnp.float32),
    ],
)


def _agg_body(g_hbm, src_hbm, dst_hbm, zeros_hbm, out_hbm,
              si_v, d0, d1, rows, acc_sh, gsems, dsems, ssems):
    c = lax.axis_index("c")
    s = lax.axis_index("s")
    wid = c * NS + s
    lo = ((wid * NCHTOT) // NW) // 8 * 8
    hi = (((wid + 1) * NCHTOT) // NW) // 8 * 8
    hi = jnp.where(wid == NW - 1, NCHTOT, hi)
    n = hi - lo

    dcur = (d0, d1)

    def dpref(cc, db):
        pltpu.async_copy(dst_hbm.at[pl.ds((lo + cc) * CH, CH)],
                         dcur[db], dsems[db])

    def dwait(cc, db):
        pltpu.make_async_copy(dst_hbm.at[pl.ds((lo + cc) * CH, CH)],
                              dcur[db], dsems[db]).wait()

    def gstart(cc, rb):
        pltpu.async_copy(g_hbm.at[si_v.at[pl.ds(cc * CH, CH)]],
                         rows.at[rb], gsems[rb])

    def gwait(cc, rb):
        pltpu.make_async_copy(g_hbm.at[si_v.at[pl.ds(cc * CH, CH)]],
                              rows.at[rb], gsems[rb]).wait()

    pltpu.sync_copy(zeros_hbm, acc_sh.at[pl.ds(s * RPS, RPS)])
    pltpu.sync_copy(src_hbm.at[pl.ds(lo * CH, MAXCH * CH)], si_v)
    plsc.subcore_barrier()

    def sstart(rb, db):
        pltpu.async_copy(rows.at[rb], acc_sh.at[dcur[db]], ssems[db],
                         add=True)

    def swait(rb, db):
        pltpu.make_async_copy(rows.at[rb], acc_sh.at[dcur[db]],
                              ssems[db]).wait()

    dpref(0, 0)
    for k in range(NB):
        gstart(k, k)

    UNROLL = 4

    def outer(i, _):
        for k in range(UNROLL):
            cc = i * UNROLL + k
            rb = k % NB
            db = k % 2

            @pl.when((cc >= 1) & (cc <= n))
            def _():
                swait((k + UNROLL - 1) % NB, 1 - db)

            @pl.when(cc + 1 < n)
            def _():
                dpref(cc + 1, 1 - db)

            @pl.when((cc >= 1) & (cc + NB - 1 < n))
            def _():
                gstart(cc + NB - 1, (k + NB - 1) % NB)

            @pl.when(cc < n)
            def _():
                gwait(cc, rb)
                dwait(cc, db)
                sstart(rb, db)
        return 0
    lax.fori_loop(0, MAXCH // UNROLL + 1, outer, 0)

    plsc.subcore_barrier()
    pltpu.sync_copy(acc_sh.at[pl.ds(s * RPS, RPS)],
                    out_hbm.at[c, pl.ds(s * RPS, RPS)])


_sc_aggregate = pl.kernel(
    _agg_body,
    out_type=jax.ShapeDtypeStruct((NC, NPAD, F), jnp.float32),
    mesh=_SC_MESH,
    scratch_types=[
        pltpu.VMEM((MAXCH * CH,), jnp.int32),
        pltpu.VMEM((CH,), jnp.int32),
        pltpu.VMEM((CH,), jnp.int32),
        pltpu.VMEM((NB, CH, F), jnp.float32),
        pltpu.VMEM_SHARED((NPAD, F), jnp.float32),
        [pltpu.SemaphoreType.DMA] * NB,
        [pltpu.SemaphoreType.DMA] * 2,
        [pltpu.SemaphoreType.DMA] * 2,
    ],
)


def _norms_body(cnt_ref, out_ref):
    deg = jnp.sum(cnt_ref[...], axis=1)
    out_ref[...] = jnp.where(deg > 0, lax.rsqrt(deg), 0.0)


def _tc_norms(counts4):
    return pl.pallas_call(
        _norms_body,
        out_shape=jax.ShapeDtypeStruct((2, NPAD // F, F), jnp.float32),
    )(counts4)


BM = 2048
GRID_M = NPAD // BM


def _mm1_body(x_ref, w_ref, no_ref, out_ref):
    out_ref[...] = jnp.dot(
        x_ref[...], w_ref[...], preferred_element_type=jnp.float32
    ) * no_ref[...]


def _tc_mm1(x, W1, norm_out):
    return pl.pallas_call(
        _mm1_body,
        grid=(GRID_M,),
        in_specs=[
            pl.BlockSpec((BM, F), lambda m: (m, 0)),
            pl.BlockSpec((F, F), lambda m: (0, 0)),
            pl.BlockSpec((BM, 1), lambda m: (m, 0)),
        ],
        out_specs=pl.BlockSpec((BM, F), lambda m: (m, 0)),
        out_shape=jax.ShapeDtypeStruct((NPAD, F), jnp.float32),
    )(x, W1, norm_out)


def _mm2_body(p_ref, ni_ref, b_ref, w_ref, no_ref, out_ref):
    agg = p_ref[0].astype(jnp.float32) + p_ref[1].astype(jnp.float32)
    h = jnp.maximum(agg * ni_ref[...] + b_ref[...], 0.0)
    out_ref[...] = jnp.dot(
        h, w_ref[...], preferred_element_type=jnp.float32
    ) * no_ref[...]


def _tc_mm2(part, norm_in, b1, W2, norm_out):
    return pl.pallas_call(
        _mm2_body,
        grid=(GRID_M,),
        in_specs=[
            pl.BlockSpec((NC, BM, F), lambda m: (0, m, 0)),
            pl.BlockSpec((BM, 1), lambda m: (m, 0)),
            pl.BlockSpec((1, F), lambda m: (0, 0)),
            pl.BlockSpec((F, F), lambda m: (0, 0)),
            pl.BlockSpec((BM, 1), lambda m: (m, 0)),
        ],
        out_specs=pl.BlockSpec((BM, F), lambda m: (m, 0)),
        out_shape=jax.ShapeDtypeStruct((NPAD, F), jnp.float32),
    )(part, norm_in, b1.reshape(1, F), W2, norm_out)


def _final_body(p_ref, ni_ref, b_ref, out_ref):
    m = pl.program_id(0)
    agg = p_ref[0].astype(jnp.float32) + p_ref[1].astype(jnp.float32)
    h = jnp.maximum(agg * ni_ref[...] + b_ref[...], 0.0)
    rows = lax.broadcasted_iota(jnp.int32, (BM, 1), 0) + m * BM
    h = jnp.where(rows < N, h, 0.0)
    part = jnp.sum(h, axis=0, keepdims=True) * (1.0 / N)

    @pl.when(m == 0)
    def _():
        out_ref[...] = part

    @pl.when(m > 0)
    def _():
        out_ref[...] += part


def _tc_final(part, norm_in, b2):
    return pl.pallas_call(
        _final_body,
        grid=(GRID_M,),
        in_specs=[
            pl.BlockSpec((NC, BM, F), lambda m: (0, m, 0)),
            pl.BlockSpec((BM, 1), lambda m: (m, 0)),
            pl.BlockSpec((1, F), lambda m: (0, 0)),
        ],
        out_specs=pl.BlockSpec((1, F), lambda m: (0, 0)),
        out_shape=jax.ShapeDtypeStruct((1, F), jnp.float32),
    )(part, norm_in, b2.reshape(1, F))


@jax.jit
def kernel(node_feats, edge_index, W1, b1, W2, b2):
    src = edge_index[0].astype(jnp.int32)
    dst = edge_index[1].astype(jnp.int32)

    zflat = jnp.zeros((NPAD,), jnp.float32)
    counts = _sc_degrees(src, dst, zflat)
    norms = _tc_norms(counts.reshape(2, NW, NPAD // F, F))
    norm_out = norms[0].reshape(NPAD, 1)
    norm_in = norms[1].reshape(NPAD, 1)

    xp = jnp.zeros((NPAD, F), jnp.float32).at[:N].set(node_feats)
    zrows = jnp.zeros((RPS, F), jnp.float32)

    g1 = _tc_mm1(xp, W1, norm_out)
    p1 = _sc_aggregate(g1, src, dst, zrows)
    g2 = _tc_mm2(p1, norm_in, b1, W2, norm_out)
    p2 = _sc_aggregate(g2, src, dst, zrows)
    return _tc_final(p2, norm_in, b2)


# speedup vs baseline: 1.1018x; 1.0017x over previous
import jax
import jax.numpy as jnp
from jax import lax
from jax.experimental import pallas as pl
from jax.experimental.pallas import tpu as pltpu
from jax.experimental.pallas import tpu_sc as plsc

N = 10000
NPAD = 10240
E = 320000
F = 128
NC = 2
NS = 16
NW = NC * NS
EPW = E // NW
CH = 64
NCHTOT = E // CH
MAXCH = 160
NB = 4
RPS = NPAD // NS

_SC_MESH = plsc.VectorSubcoreMesh(core_axis_name="c", subcore_axis_name="s")


def _degrees_body(src_hbm, dst_hbm, zflat_hbm, out_hbm, idx_v, cnt_v):
    c = lax.axis_index("c")
    s = lax.axis_index("s")
    wid = c * NS + s
    ones = jnp.ones((16,), jnp.float32)
    for kind, edge_hbm in ((0, src_hbm), (1, dst_hbm)):
        pltpu.sync_copy(zflat_hbm, cnt_v)
        pltpu.sync_copy(edge_hbm.at[pl.ds(wid * EPW, EPW)], idx_v)

        def count_step(j, _):
            for q in range(4):
                idx = idx_v[pl.ds(j * 64 + q * 16, 16)]
                plsc.addupdate_scatter(cnt_v, [idx], ones)
            return 0
        lax.fori_loop(0, EPW // 64, count_step, 0)
        for r in range(EPW // 64 * 64, EPW, 16):
            plsc.addupdate_scatter(cnt_v, [idx_v[pl.ds(r, 16)]], ones)
        pltpu.sync_copy(cnt_v, out_hbm.at[kind, wid])


_sc_degrees = pl.kernel(
    _degrees_body,
    out_type=jax.ShapeDtypeStruct((2, NW, NPAD), jnp.float32),
    mesh=_SC_MESH,
    compiler_params=pltpu.CompilerParams(needs_layout_passes=False),
    scratch_types=[
        pltpu.VMEM((EPW,), jnp.int32),
        pltpu.VMEM((NPAD,), jnp.float32),
    ],
)


def _agg_body(g_hbm, src_hbm, dst_hbm, zeros_hbm, out_hbm,
              si_v, d0, d1, rows, acc_sh, gsems, dsems, ssems):
    c = lax.axis_index("c")
    s = lax.axis_index("s")
    wid = c * NS + s
    lo = ((wid * NCHTOT) // NW) // 8 * 8
    hi = (((wid + 1) * NCHTOT) // NW) // 8 * 8
    hi = jnp.where(wid == NW - 1, NCHTOT, hi)
    n = hi - lo

    dcur = (d0, d1)

    def dpref(cc, db):
        pltpu.async_copy(dst_hbm.at[pl.ds((lo + cc) * CH, CH)],
                         dcur[db], dsems[db])

    def dwait(cc, db):
        pltpu.make_async_copy(dst_hbm.at[pl.ds((lo + cc) * CH, CH)],
                              dcur[db], dsems[db]).wait()

    def gstart(cc, rb):
        pltpu.async_copy(g_hbm.at[si_v.at[pl.ds(cc * CH, CH)]],
                         rows.at[rb], gsems[rb])

    def gwait(cc, rb):
        pltpu.make_async_copy(g_hbm.at[si_v.at[pl.ds(cc * CH, CH)]],
                              rows.at[rb], gsems[rb]).wait()

    pltpu.sync_copy(zeros_hbm, acc_sh.at[pl.ds(s * RPS, RPS)])
    pltpu.sync_copy(src_hbm.at[pl.ds(lo * CH, MAXCH * CH)], si_v)
    plsc.subcore_barrier()

    def sstart(rb, db):
        pltpu.async_copy(rows.at[rb], acc_sh.at[dcur[db]], ssems[db],
                         add=True)

    def swait(rb, db):
        pltpu.make_async_copy(rows.at[rb], acc_sh.at[dcur[db]],
                              ssems[db]).wait()

    dpref(0, 0)
    for k in range(NB):
        gstart(k, k)

    UNROLL = 4

    def outer(i, _):
        for k in range(UNROLL):
            cc = i * UNROLL + k
            rb = k % NB
            db = k % 2

            @pl.when((cc >= 1) & (cc <= n))
            def _():
                swait((k + UNROLL - 1) % NB, 1 - db)

            @pl.when(cc + 1 < n)
            def _():
                dpref(cc + 1, 1 - db)

            @pl.when((cc >= 1) & (cc + NB - 1 < n))
            def _():
                gstart(cc + NB - 1, (k + NB - 1) % NB)

            @pl.when(cc < n)
            def _():
                gwait(cc, rb)
                dwait(cc, db)
                sstart(rb, db)
        return 0
    lax.fori_loop(0, MAXCH // UNROLL + 1, outer, 0)

    plsc.subcore_barrier()
    pltpu.sync_copy(acc_sh.at[pl.ds(s * RPS, RPS)],
                    out_hbm.at[c, pl.ds(s * RPS, RPS)])


_sc_aggregate = pl.kernel(
    _agg_body,
    out_type=jax.ShapeDtypeStruct((NC, NPAD, F), jnp.float32),
    mesh=_SC_MESH,
    scratch_types=[
        pltpu.VMEM((MAXCH * CH,), jnp.int32),
        pltpu.VMEM((CH,), jnp.int32),
        pltpu.VMEM((CH,), jnp.int32),
        pltpu.VMEM((NB, CH, F), jnp.float32),
        pltpu.VMEM_SHARED((NPAD, F), jnp.float32),
        [pltpu.SemaphoreType.DMA] * NB,
        [pltpu.SemaphoreType.DMA] * 2,
        [pltpu.SemaphoreType.DMA] * 2,
    ],
)


def _norms_body(cnt_ref, out_ref):
    deg = jnp.sum(cnt_ref[...], axis=1)
    out_ref[...] = jnp.where(deg > 0, lax.rsqrt(deg), 0.0)


def _tc_norms(counts4):
    return pl.pallas_call(
        _norms_body,
        out_shape=jax.ShapeDtypeStruct((2, NPAD // F, F), jnp.float32),
    )(counts4)


BM = 2048
GRID_M = NPAD // BM


def _mm1_body(x_ref, w_ref, no_ref, out_ref):
    out_ref[...] = jnp.dot(
        x_ref[...], w_ref[...], preferred_element_type=jnp.float32
    ) * no_ref[...]


def _tc_mm1(x, W1, norm_out):
    return pl.pallas_call(
        _mm1_body,
        grid=(GRID_M,),
        in_specs=[
            pl.BlockSpec((BM, F), lambda m: (m, 0)),
            pl.BlockSpec((F, F), lambda m: (0, 0)),
            pl.BlockSpec((BM, 1), lambda m: (m, 0)),
        ],
        out_specs=pl.BlockSpec((BM, F), lambda m: (m, 0)),
        out_shape=jax.ShapeDtypeStruct((NPAD, F), jnp.float32),
    )(x, W1, norm_out)


def _mm2_body(p_ref, ni_ref, b_ref, w_ref, no_ref, out_ref):
    agg = p_ref[0].astype(jnp.float32) + p_ref[1].astype(jnp.float32)
    h = jnp.maximum(agg * ni_ref[...] + b_ref[...], 0.0)
    out_ref[...] = jnp.dot(
        h, w_ref[...], preferred_element_type=jnp.float32
    ) * no_ref[...]


def _tc_mm2(part, norm_in, b1, W2, norm_out):
    return pl.pallas_call(
        _mm2_body,
        grid=(GRID_M,),
        in_specs=[
            pl.BlockSpec((NC, BM, F), lambda m: (0, m, 0)),
            pl.BlockSpec((BM, 1), lambda m: (m, 0)),
            pl.BlockSpec((1, F), lambda m: (0, 0)),
            pl.BlockSpec((F, F), lambda m: (0, 0)),
            pl.BlockSpec((BM, 1), lambda m: (m, 0)),
        ],
        out_specs=pl.BlockSpec((BM, F), lambda m: (m, 0)),
        out_shape=jax.ShapeDtypeStruct((NPAD, F), jnp.float32),
    )(part, norm_in, b1.reshape(1, F), W2, norm_out)


def _final_body(p_ref, ni_ref, b_ref, out_ref):
    m = pl.program_id(0)
    agg = p_ref[0].astype(jnp.float32) + p_ref[1].astype(jnp.float32)
    h = jnp.maximum(agg * ni_ref[...] + b_ref[...], 0.0)
    rows = lax.broadcasted_iota(jnp.int32, (BM, 1), 0) + m * BM
    h = jnp.where(rows < N, h, 0.0)
    part = jnp.sum(h, axis=0, keepdims=True) * (1.0 / N)

    @pl.when(m == 0)
    def _():
        out_ref[...] = part

    @pl.when(m > 0)
    def _():
        out_ref[...] += part


def _tc_final(part, norm_in, b2):
    return pl.pallas_call(
        _final_body,
        grid=(GRID_M,),
        in_specs=[
            pl.BlockSpec((NC, BM, F), lambda m: (0, m, 0)),
            pl.BlockSpec((BM, 1), lambda m: (m, 0)),
            pl.BlockSpec((1, F), lambda m: (0, 0)),
        ],
        out_specs=pl.BlockSpec((1, F), lambda m: (0, 0)),
        out_shape=jax.ShapeDtypeStruct((1, F), jnp.float32),
    )(part, norm_in, b2.reshape(1, F))


@jax.jit
def kernel(node_feats, edge_index, W1, b1, W2, b2):
    src = edge_index[0].astype(jnp.int32)
    dst = edge_index[1].astype(jnp.int32)

    zflat = jnp.zeros((NPAD,), jnp.float32)
    counts = _sc_degrees(src, dst, zflat)
    norms = _tc_norms(counts.reshape(2, NW, NPAD // F, F))
    norm_out = norms[0].reshape(NPAD, 1)
    norm_in = norms[1].reshape(NPAD, 1)

    xp = jnp.zeros((NPAD, F), jnp.float32).at[:N].set(node_feats)
    zrows = jnp.zeros((RPS, F), jnp.float32)

    g1 = _tc_mm1(xp, W1, norm_out)
    p1 = _sc_aggregate(g1, src, dst, zrows)
    g2 = _tc_mm2(p1, norm_in, b1, W2, norm_out)
    p2 = _sc_aggregate(g2, src, dst, zrows)
    return _tc_final(p2, norm_in, b2)
